# baseline (speedup 1.0000x reference)
import jax
import jax.numpy as jnp
import numpy as np
from jax import lax
from jax.experimental import pallas as pl
from jax.experimental.pallas import tpu as pltpu
from jax.experimental.pallas import tpu_sc as plsc

L = 16
F = 2
T = 2 ** 19
N_MIN = 16
N_MAX = 2048
N_PTS = 262144
B_SCALE = float(np.exp((np.log(float(N_MAX)) - np.log(float(N_MIN))) / (L - 1)))
P1 = np.uint32(2654435761)
P2 = np.uint32(805459861)

RES = np.array([np.floor(N_MIN * (B_SCALE ** l)) for l in range(L)], dtype=np.float32)

NC = 2
NS = 16
NW = NC * NS
PTS_PER_TILE = N_PTS // NW
C = 512
NCH = PTS_PER_TILE // C
NPAIR = NCH // 2
G16 = C // 16


def _tec_body(xa_hbm, xb_hbm, xc_hbm, taba_hbm, tabb_hbm, res_hbm, out_hbm,
              xv, resv, sla, slb, idx0, idx1, fa0, fa1, fb0, fb1, outv,
              sa0, sb0, sa1, sb1):
    wid = lax.axis_index("s") * NC + lax.axis_index("c")
    sid = lax.axis_index("s")
    idxb = (idx0, idx1)
    fab = (fa0, fa1)
    fbb = (fb0, fb1)
    sab = (sa0, sa1)
    sbb = (sb0, sb1)
    tbase = wid * PTS_PER_TILE

    pltpu.sync_copy(xa_hbm.at[pl.ds(tbase, PTS_PER_TILE)],
                    xv.at[pl.ds(0, PTS_PER_TILE)])
    pltpu.sync_copy(xb_hbm.at[pl.ds(tbase, PTS_PER_TILE)],
                    xv.at[pl.ds(PTS_PER_TILE, PTS_PER_TILE)])
    pltpu.sync_copy(xc_hbm.at[pl.ds(tbase, PTS_PER_TILE)],
                    xv.at[pl.ds(2 * PTS_PER_TILE, PTS_PER_TILE)])
    pltpu.sync_copy(res_hbm, resv)
    resvec = resv[...]
    zero16 = lax.iota(jnp.int32, 16) * 0

    def fire(l, ch, par, res):
        idx = idxb[par]

        def idx_body(g, _):
            p = g * 16
            xs0 = xv[pl.ds(ch * C + p, 16)] * res
            xs1 = xv[pl.ds(PTS_PER_TILE + ch * C + p, 16)] * res
            xs2 = xv[pl.ds(2 * PTS_PER_TILE + ch * C + p, 16)] * res
            u0 = xs0.astype(jnp.int32).astype(jnp.uint32)
            a1 = xs1.astype(jnp.int32).astype(jnp.uint32) * P1
            a2 = xs2.astype(jnp.int32).astype(jnp.uint32) * P2
            a0b = u0 + jnp.uint32(1)
            a1b = a1 + P1
            a2b = a2 + P2
            for k in range(8):
                h = ((a0b if (k >> 2) & 1 else u0)
                     ^ (a1b if (k >> 1) & 1 else a1)
                     ^ (a2b if k & 1 else a2))
                hm = (h & jnp.uint32(T - 1)).astype(jnp.int32)
                idx[pl.ds(k * C + p, 16)] = hm
            return 0

        lax.fori_loop(0, G16, idx_body, 0)
        pltpu.async_copy(sla.at[idx], fab[par], sab[par])

    def fire_b_hbm(l, par):
        idx = idxb[par]
        pltpu.async_copy(tabb_hbm.at[pl.ds(l * T, T)].at[idx],
                         fbb[par], sbb[par])

    def fire_b_sp(par):
        idx = idxb[par]
        pltpu.async_copy(slb.at[idx], fbb[par], sbb[par])

    def wait(par):
        pltpu.make_async_copy(taba_hbm.at[pl.ds(0, 8 * C)],
                              fab[par], sab[par]).wait()
        pltpu.make_async_copy(tabb_hbm.at[pl.ds(0, 8 * C)],
                              fbb[par], sbb[par]).wait()

    def fma(l, ch, par, res):
        fa = fab[par]
        fb = fbb[par]

        def fma_body(g, _):
            p = g * 16
            xs0 = xv[pl.ds(ch * C + p, 16)] * res
            xs1 = xv[pl.ds(PTS_PER_TILE + ch * C + p, 16)] * res
            xs2 = xv[pl.ds(2 * PTS_PER_TILE + ch * C + p, 16)] * res
            w0 = xs0 - xs0.astype(jnp.int32).astype(jnp.float32)
            w1 = xs1 - xs1.astype(jnp.int32).astype(jnp.float32)
            w2 = xs2 - xs2.astype(jnp.int32).astype(jnp.float32)
            v0 = 1.0 - w0
            v1 = 1.0 - w1
            v2 = 1.0 - w2
            q0 = v1 * v2
            q1 = v1 * w2
            q2 = w1 * v2
            q3 = w1 * w2
            acc0 = jnp.zeros((16,), jnp.float32)
            acc1 = jnp.zeros((16,), jnp.float32)
            for k in range(8):
                qq = (q0, q1, q2, q3)[k & 3]
                wk = (w0 if (k >> 2) & 1 else v0) * qq
                sl = pl.ds(k * C + p, 16)
                acc0 = acc0 + wk * fa[sl]
                acc1 = acc1 + wk * fb[sl]
            outv[pl.ds(p, 16)] = acc0
            outv[pl.ds(C + p, 16)] = acc1
            return 0

        lax.fori_loop(0, G16, fma_body, 0)
        base = tbase + ch * C
        pltpu.sync_copy(outv.at[pl.ds(0, C)],
                        out_hbm.at[pl.ds(2 * l * N_PTS + base, C)])
        pltpu.sync_copy(outv.at[pl.ds(C, C)],
                        out_hbm.at[pl.ds((2 * l + 1) * N_PTS + base, C)])

    def level_body(l, _):
        res = resvec.at[zero16 + l].get(mode="promise_in_bounds")
        plsc.subcore_barrier()

        @pl.when(sid == 0)
        def _stage_a():
            pltpu.sync_copy(taba_hbm.at[pl.ds(l * T, T)], sla)

        @pl.when(sid == 1)
        def _stage_b():
            pltpu.sync_copy(tabb_hbm.at[pl.ds(l * T, T)], slb)

        plsc.subcore_barrier()

        fire(l, 0, 0, res)
        fire_b_hbm(l, 0)

        def pair_body(i, _):
            fire(l, 2 * i + 1, 1, res)
            third = i - (i // 3) * 3 == 2

            @pl.when(third)
            def _b1h():
                fire_b_hbm(l, 1)

            @pl.when(jnp.logical_not(third))
            def _b1s():
                fire_b_sp(1)

            wait(0)
            fma(l, 2 * i, 0, res)

            @pl.when(i < NPAIR - 1)
            def _fire_next():
                fire(l, 2 * i + 2, 0, res)
                fire_b_hbm(l, 0)

            wait(1)
            fma(l, 2 * i + 1, 1, res)
            return 0

        lax.fori_loop(0, NPAIR, pair_body, 0)
        return 0

    lax.fori_loop(0, L, level_body, 0)


def _tr_body(*refs):
    inp = refs[:L * F]
    out_ref = refs[L * F]
    out_ref[...] = jnp.transpose(jnp.stack([r[...] for r in inp], axis=0), (1, 0))


TB = 2048


@jax.jit
def _encode(xa, xb, xc, taba, tabb):
    res_in = jnp.asarray(RES)
    mesh = plsc.VectorSubcoreMesh(core_axis_name="c", subcore_axis_name="s")
    k = pl.kernel(
        _tec_body,
        out_type=jax.ShapeDtypeStruct((L * F * N_PTS,), jnp.float32),
        mesh=mesh,
        scratch_types=[
            pltpu.VMEM((3 * PTS_PER_TILE,), jnp.float32),
            pltpu.VMEM((L,), jnp.float32),
            pltpu.VMEM_SHARED((T,), jnp.float32),
            pltpu.VMEM_SHARED((T,), jnp.float32),
            pltpu.VMEM((8 * C,), jnp.int32),
            pltpu.VMEM((8 * C,), jnp.int32),
            pltpu.VMEM((8 * C,), jnp.float32),
            pltpu.VMEM((8 * C,), jnp.float32),
            pltpu.VMEM((8 * C,), jnp.float32),
            pltpu.VMEM((8 * C,), jnp.float32),
            pltpu.VMEM((F * C,), jnp.float32),
            pltpu.SemaphoreType.DMA,
            pltpu.SemaphoreType.DMA,
            pltpu.SemaphoreType.DMA,
            pltpu.SemaphoreType.DMA,
        ],
    )
    out_lm = k(xa, xb, xc, taba, tabb, res_in)
    nb = N_PTS // TB
    out = pl.pallas_call(
        _tr_body,
        out_shape=jax.ShapeDtypeStruct((N_PTS, L * F), jnp.float32),
        grid=(nb,),
        in_specs=[pl.BlockSpec((TB,), lambda i, r=r: (r * nb + i,))
                  for r in range(L * F)],
        out_specs=pl.BlockSpec((TB, L * F), lambda i: (i, 0)),
    )(*([out_lm] * (L * F)))
    return out


@jax.jit
def _prep(x, table):
    return (x[:, 0], x[:, 1], x[:, 2],
            table[:, :, 0].reshape(L * T), table[:, :, 1].reshape(L * T))


def kernel(x, table):
    xa, xb, xc, taba, tabb = _prep(x, table)
    return _encode(xa, xb, xc, taba, tabb)


# speedup vs baseline: 1.0903x; 1.0903x over previous
import jax
import jax.numpy as jnp
import numpy as np
from jax import lax
from jax.experimental import pallas as pl
from jax.experimental.pallas import tpu as pltpu
from jax.experimental.pallas import tpu_sc as plsc

L = 16
F = 2
T = 2 ** 19
N_MIN = 16
N_MAX = 2048
N_PTS = 262144
B_SCALE = float(np.exp((np.log(float(N_MAX)) - np.log(float(N_MIN))) / (L - 1)))
P1 = np.uint32(2654435761)
P2 = np.uint32(805459861)

RES = np.array([np.floor(N_MIN * (B_SCALE ** l)) for l in range(L)], dtype=np.float32)

NC = 2
NS = 16
NW = NC * NS
PTS_PER_TILE = N_PTS // NW
C = 512
NCH = PTS_PER_TILE // C
NPAIR = NCH // 2
G16 = C // 16


def _tec_body(xa_hbm, xb_hbm, xc_hbm, taba_hbm, tabb_hbm, res_hbm, out_hbm,
              xv, resv, sla, slb, idx0, idx1, fa0, fa1, fb0, fb1, outv,
              sa0, sb0, sa1, sb1):
    wid = lax.axis_index("s") * NC + lax.axis_index("c")
    sid = lax.axis_index("s")
    idxb = (idx0, idx1)
    fab = (fa0, fa1)
    fbb = (fb0, fb1)
    sab = (sa0, sa1)
    sbb = (sb0, sb1)
    tbase = wid * PTS_PER_TILE

    pltpu.sync_copy(xa_hbm.at[pl.ds(tbase, PTS_PER_TILE)],
                    xv.at[pl.ds(0, PTS_PER_TILE)])
    pltpu.sync_copy(xb_hbm.at[pl.ds(tbase, PTS_PER_TILE)],
                    xv.at[pl.ds(PTS_PER_TILE, PTS_PER_TILE)])
    pltpu.sync_copy(xc_hbm.at[pl.ds(tbase, PTS_PER_TILE)],
                    xv.at[pl.ds(2 * PTS_PER_TILE, PTS_PER_TILE)])
    pltpu.sync_copy(res_hbm, resv)
    resvec = resv[...]
    zero16 = lax.iota(jnp.int32, 16) * 0

    def fire(l, ch, par, res):
        idx = idxb[par]

        def idx_body(g, _):
            p = g * 16
            xs0 = xv[pl.ds(ch * C + p, 16)] * res
            xs1 = xv[pl.ds(PTS_PER_TILE + ch * C + p, 16)] * res
            xs2 = xv[pl.ds(2 * PTS_PER_TILE + ch * C + p, 16)] * res
            u0 = xs0.astype(jnp.int32).astype(jnp.uint32)
            a1 = xs1.astype(jnp.int32).astype(jnp.uint32) * P1
            a2 = xs2.astype(jnp.int32).astype(jnp.uint32) * P2
            a0b = u0 + jnp.uint32(1)
            a1b = a1 + P1
            a2b = a2 + P2
            for k in range(8):
                h = ((a0b if (k >> 2) & 1 else u0)
                     ^ (a1b if (k >> 1) & 1 else a1)
                     ^ (a2b if k & 1 else a2))
                hm = (h & jnp.uint32(T - 1)).astype(jnp.int32)
                idx[pl.ds(k * C + p, 16)] = hm
            return 0

        lax.fori_loop(0, G16, idx_body, 0)
        pltpu.async_copy(sla.at[idx], fab[par], sab[par])

    def fire_b_hbm(l, par):
        idx = idxb[par]
        pltpu.async_copy(tabb_hbm.at[pl.ds(l * T, T)].at[idx],
                         fbb[par], sbb[par])

    def fire_b_sp(par):
        idx = idxb[par]
        pltpu.async_copy(slb.at[idx], fbb[par], sbb[par])

    def wait(par):
        pltpu.make_async_copy(taba_hbm.at[pl.ds(0, 8 * C)],
                              fab[par], sab[par]).wait()
        pltpu.make_async_copy(tabb_hbm.at[pl.ds(0, 8 * C)],
                              fbb[par], sbb[par]).wait()

    def fma(l, ch, par, res):
        fa = fab[par]
        fb = fbb[par]

        def fma_body(g, _):
            p = g * 16
            xs0 = xv[pl.ds(ch * C + p, 16)] * res
            xs1 = xv[pl.ds(PTS_PER_TILE + ch * C + p, 16)] * res
            xs2 = xv[pl.ds(2 * PTS_PER_TILE + ch * C + p, 16)] * res
            w0 = xs0 - xs0.astype(jnp.int32).astype(jnp.float32)
            w1 = xs1 - xs1.astype(jnp.int32).astype(jnp.float32)
            w2 = xs2 - xs2.astype(jnp.int32).astype(jnp.float32)
            v0 = 1.0 - w0
            v1 = 1.0 - w1
            v2 = 1.0 - w2
            q0 = v1 * v2
            q1 = v1 * w2
            q2 = w1 * v2
            q3 = w1 * w2
            acc0 = jnp.zeros((16,), jnp.float32)
            acc1 = jnp.zeros((16,), jnp.float32)
            for k in range(8):
                qq = (q0, q1, q2, q3)[k & 3]
                wk = (w0 if (k >> 2) & 1 else v0) * qq
                sl = pl.ds(k * C + p, 16)
                acc0 = acc0 + wk * fa[sl]
                acc1 = acc1 + wk * fb[sl]
            outv[pl.ds(p, 16)] = acc0
            outv[pl.ds(C + p, 16)] = acc1
            return 0

        lax.fori_loop(0, G16, fma_body, 0)
        base = tbase + ch * C
        pltpu.sync_copy(outv.at[pl.ds(0, C)],
                        out_hbm.at[pl.ds(2 * l * N_PTS + base, C)])
        pltpu.sync_copy(outv.at[pl.ds(C, C)],
                        out_hbm.at[pl.ds((2 * l + 1) * N_PTS + base, C)])

    def level_body(l, _):
        res = resvec.at[zero16 + l].get(mode="promise_in_bounds")
        plsc.subcore_barrier()

        @pl.when(sid == 0)
        def _stage_a():
            pltpu.sync_copy(taba_hbm.at[pl.ds(l * T, T)], sla)

        @pl.when(sid == 1)
        def _stage_b():
            pltpu.sync_copy(tabb_hbm.at[pl.ds(l * T, T)], slb)

        plsc.subcore_barrier()

        fire(l, 0, 0, res)
        fire_b_hbm(l, 0)

        def pair_body(i, _):
            fire(l, 2 * i + 1, 1, res)
            fire_b_sp(1)
            wait(0)
            fma(l, 2 * i, 0, res)

            @pl.when(i < NPAIR - 1)
            def _fire_next():
                fire(l, 2 * i + 2, 0, res)
                fire_b_hbm(l, 0)

            wait(1)
            fma(l, 2 * i + 1, 1, res)
            return 0

        lax.fori_loop(0, NPAIR, pair_body, 0)
        return 0

    lax.fori_loop(0, L, level_body, 0)


def _tr_body(*refs):
    inp = refs[:L * F]
    out_ref = refs[L * F]
    out_ref[...] = jnp.transpose(jnp.stack([r[...] for r in inp], axis=0), (1, 0))


TB = 2048


@jax.jit
def _encode(xa, xb, xc, taba, tabb):
    res_in = jnp.asarray(RES)
    mesh = plsc.VectorSubcoreMesh(core_axis_name="c", subcore_axis_name="s")
    k = pl.kernel(
        _tec_body,
        out_type=jax.ShapeDtypeStruct((L * F * N_PTS,), jnp.float32),
        mesh=mesh,
        scratch_types=[
            pltpu.VMEM((3 * PTS_PER_TILE,), jnp.float32),
            pltpu.VMEM((L,), jnp.float32),
            pltpu.VMEM_SHARED((T,), jnp.float32),
            pltpu.VMEM_SHARED((T,), jnp.float32),
            pltpu.VMEM((8 * C,), jnp.int32),
            pltpu.VMEM((8 * C,), jnp.int32),
            pltpu.VMEM((8 * C,), jnp.float32),
            pltpu.VMEM((8 * C,), jnp.float32),
            pltpu.VMEM((8 * C,), jnp.float32),
            pltpu.VMEM((8 * C,), jnp.float32),
            pltpu.VMEM((F * C,), jnp.float32),
            pltpu.SemaphoreType.DMA,
            pltpu.SemaphoreType.DMA,
            pltpu.SemaphoreType.DMA,
            pltpu.SemaphoreType.DMA,
        ],
    )
    out_lm = k(xa, xb, xc, taba, tabb, res_in)
    nb = N_PTS // TB
    out = pl.pallas_call(
        _tr_body,
        out_shape=jax.ShapeDtypeStruct((N_PTS, L * F), jnp.float32),
        grid=(nb,),
        in_specs=[pl.BlockSpec((TB,), lambda i, r=r: (r * nb + i,))
                  for r in range(L * F)],
        out_specs=pl.BlockSpec((TB, L * F), lambda i: (i, 0)),
    )(*([out_lm] * (L * F)))
    return out


@jax.jit
def _prep(x, table):
    return (x[:, 0], x[:, 1], x[:, 2],
            table[:, :, 0].reshape(L * T), table[:, :, 1].reshape(L * T))


def kernel(x, table):
    xa, xb, xc, taba, tabb = _prep(x, table)
    return _encode(xa, xb, xc, taba, tabb)


# speedup vs baseline: 1.1100x; 1.0180x over previous
import jax
import jax.numpy as jnp
import numpy as np
from jax import lax
from jax.experimental import pallas as pl
from jax.experimental.pallas import tpu as pltpu
from jax.experimental.pallas import tpu_sc as plsc

L = 16
F = 2
T = 2 ** 19
N_MIN = 16
N_MAX = 2048
N_PTS = 262144
B_SCALE = float(np.exp((np.log(float(N_MAX)) - np.log(float(N_MIN))) / (L - 1)))
P1 = np.uint32(2654435761)
P2 = np.uint32(805459861)

RES = np.array([np.floor(N_MIN * (B_SCALE ** l)) for l in range(L)], dtype=np.float32)

NC = 2
NS = 16
NW = NC * NS
PTS_PER_TILE = N_PTS // NW
C = 512
NCH = PTS_PER_TILE // C
NPAIR = NCH // 2
G16 = C // 16


def _tec_body(xa_hbm, xb_hbm, xc_hbm, taba_hbm, tabb_hbm, res_hbm, out_hbm,
              xv, resv, sla, slb, idx0, idx1, fa0, fa1, fb0, fb1, ov0, ov1,
              sa0, sb0, sa1, sb1, so0, so1):
    wid = lax.axis_index("s") * NC + lax.axis_index("c")
    sid = lax.axis_index("s")
    idxb = (idx0, idx1)
    fab = (fa0, fa1)
    fbb = (fb0, fb1)
    sab = (sa0, sa1)
    sbb = (sb0, sb1)
    ovb = (ov0, ov1)
    sob = (so0, so1)
    tbase = wid * PTS_PER_TILE

    pltpu.sync_copy(xa_hbm.at[pl.ds(tbase, PTS_PER_TILE)],
                    xv.at[pl.ds(0, PTS_PER_TILE)])
    pltpu.sync_copy(xb_hbm.at[pl.ds(tbase, PTS_PER_TILE)],
                    xv.at[pl.ds(PTS_PER_TILE, PTS_PER_TILE)])
    pltpu.sync_copy(xc_hbm.at[pl.ds(tbase, PTS_PER_TILE)],
                    xv.at[pl.ds(2 * PTS_PER_TILE, PTS_PER_TILE)])
    pltpu.sync_copy(res_hbm, resv)
    resvec = resv[...]
    zero16 = lax.iota(jnp.int32, 16) * 0

    def fire(l, ch, par, res):
        idx = idxb[par]

        def idx_body(g, _):
            p = g * 16
            xs0 = xv[pl.ds(ch * C + p, 16)] * res
            xs1 = xv[pl.ds(PTS_PER_TILE + ch * C + p, 16)] * res
            xs2 = xv[pl.ds(2 * PTS_PER_TILE + ch * C + p, 16)] * res
            u0 = xs0.astype(jnp.int32).astype(jnp.uint32)
            a1 = xs1.astype(jnp.int32).astype(jnp.uint32) * P1
            a2 = xs2.astype(jnp.int32).astype(jnp.uint32) * P2
            a0b = u0 + jnp.uint32(1)
            a1b = a1 + P1
            a2b = a2 + P2
            for k in range(8):
                h = ((a0b if (k >> 2) & 1 else u0)
                     ^ (a1b if (k >> 1) & 1 else a1)
                     ^ (a2b if k & 1 else a2))
                hm = (h & jnp.uint32(T - 1)).astype(jnp.int32)
                idx[pl.ds(k * C + p, 16)] = hm
            return 0

        lax.fori_loop(0, G16, idx_body, 0)
        pltpu.async_copy(sla.at[idx], fab[par], sab[par])

    def fire_b_hbm(l, par):
        idx = idxb[par]
        pltpu.async_copy(tabb_hbm.at[pl.ds(l * T, T)].at[idx],
                         fbb[par], sbb[par])

    def fire_b_sp(par):
        idx = idxb[par]
        pltpu.async_copy(slb.at[idx], fbb[par], sbb[par])

    def wait(par):
        pltpu.make_async_copy(taba_hbm.at[pl.ds(0, 8 * C)],
                              fab[par], sab[par]).wait()
        pltpu.make_async_copy(tabb_hbm.at[pl.ds(0, 8 * C)],
                              fbb[par], sbb[par]).wait()

    def fma(l, ch, par, res):
        fa = fab[par]
        fb = fbb[par]
        outv = ovb[par]

        @pl.when(jnp.logical_or(l > 0, ch >= 2))
        def _drain_out():
            pltpu.make_async_copy(outv.at[pl.ds(0, C)],
                                  out_hbm.at[pl.ds(0, C)], sob[par]).wait()
            pltpu.make_async_copy(outv.at[pl.ds(C, C)],
                                  out_hbm.at[pl.ds(0, C)], sob[par]).wait()

        def fma_body(g, _):
            p = g * 16
            xs0 = xv[pl.ds(ch * C + p, 16)] * res
            xs1 = xv[pl.ds(PTS_PER_TILE + ch * C + p, 16)] * res
            xs2 = xv[pl.ds(2 * PTS_PER_TILE + ch * C + p, 16)] * res
            w0 = xs0 - xs0.astype(jnp.int32).astype(jnp.float32)
            w1 = xs1 - xs1.astype(jnp.int32).astype(jnp.float32)
            w2 = xs2 - xs2.astype(jnp.int32).astype(jnp.float32)
            v0 = 1.0 - w0
            v1 = 1.0 - w1
            v2 = 1.0 - w2
            q0 = v1 * v2
            q1 = v1 * w2
            q2 = w1 * v2
            q3 = w1 * w2
            acc0 = jnp.zeros((16,), jnp.float32)
            acc1 = jnp.zeros((16,), jnp.float32)
            for k in range(8):
                qq = (q0, q1, q2, q3)[k & 3]
                wk = (w0 if (k >> 2) & 1 else v0) * qq
                sl = pl.ds(k * C + p, 16)
                acc0 = acc0 + wk * fa[sl]
                acc1 = acc1 + wk * fb[sl]
            outv[pl.ds(p, 16)] = acc0
            outv[pl.ds(C + p, 16)] = acc1
            return 0

        lax.fori_loop(0, G16, fma_body, 0)
        base = tbase + ch * C
        pltpu.async_copy(outv.at[pl.ds(0, C)],
                         out_hbm.at[pl.ds(2 * l * N_PTS + base, C)], sob[par])
        pltpu.async_copy(outv.at[pl.ds(C, C)],
                         out_hbm.at[pl.ds((2 * l + 1) * N_PTS + base, C)],
                         sob[par])

    def level_body(l, _):
        res = resvec.at[zero16 + l].get(mode="promise_in_bounds")
        plsc.subcore_barrier()

        @pl.when(sid == 0)
        def _stage_a():
            pltpu.sync_copy(taba_hbm.at[pl.ds(l * T, T)], sla)

        @pl.when(sid == 1)
        def _stage_b():
            pltpu.sync_copy(tabb_hbm.at[pl.ds(l * T, T)], slb)

        plsc.subcore_barrier()

        fire(l, 0, 0, res)
        fire_b_hbm(l, 0)

        def pair_body(i, _):
            fire(l, 2 * i + 1, 1, res)
            fire_b_sp(1)
            wait(0)
            fma(l, 2 * i, 0, res)

            @pl.when(i < NPAIR - 1)
            def _fire_next():
                fire(l, 2 * i + 2, 0, res)
                fire_b_hbm(l, 0)

            wait(1)
            fma(l, 2 * i + 1, 1, res)
            return 0

        lax.fori_loop(0, NPAIR, pair_body, 0)
        return 0

    lax.fori_loop(0, L, level_body, 0)
    for par in (0, 1):
        pltpu.make_async_copy(ovb[par].at[pl.ds(0, C)],
                              out_hbm.at[pl.ds(0, C)], sob[par]).wait()
        pltpu.make_async_copy(ovb[par].at[pl.ds(C, C)],
                              out_hbm.at[pl.ds(0, C)], sob[par]).wait()


def _tr_body(*refs):
    inp = refs[:L * F]
    out_ref = refs[L * F]
    out_ref[...] = jnp.transpose(jnp.stack([r[...] for r in inp], axis=0), (1, 0))


TB = 2048


@jax.jit
def _encode(xa, xb, xc, taba, tabb):
    res_in = jnp.asarray(RES)
    mesh = plsc.VectorSubcoreMesh(core_axis_name="c", subcore_axis_name="s")
    k = pl.kernel(
        _tec_body,
        out_type=jax.ShapeDtypeStruct((L * F * N_PTS,), jnp.float32),
        mesh=mesh,
        scratch_types=[
            pltpu.VMEM((3 * PTS_PER_TILE,), jnp.float32),
            pltpu.VMEM((L,), jnp.float32),
            pltpu.VMEM_SHARED((T,), jnp.float32),
            pltpu.VMEM_SHARED((T,), jnp.float32),
            pltpu.VMEM((8 * C,), jnp.int32),
            pltpu.VMEM((8 * C,), jnp.int32),
            pltpu.VMEM((8 * C,), jnp.float32),
            pltpu.VMEM((8 * C,), jnp.float32),
            pltpu.VMEM((8 * C,), jnp.float32),
            pltpu.VMEM((8 * C,), jnp.float32),
            pltpu.VMEM((F * C,), jnp.float32),
            pltpu.VMEM((F * C,), jnp.float32),
            pltpu.SemaphoreType.DMA,
            pltpu.SemaphoreType.DMA,
            pltpu.SemaphoreType.DMA,
            pltpu.SemaphoreType.DMA,
            pltpu.SemaphoreType.DMA,
            pltpu.SemaphoreType.DMA,
        ],
    )
    out_lm = k(xa, xb, xc, taba, tabb, res_in)
    nb = N_PTS // TB
    out = pl.pallas_call(
        _tr_body,
        out_shape=jax.ShapeDtypeStruct((N_PTS, L * F), jnp.float32),
        grid=(nb,),
        in_specs=[pl.BlockSpec((TB,), lambda i, r=r: (r * nb + i,))
                  for r in range(L * F)],
        out_specs=pl.BlockSpec((TB, L * F), lambda i: (i, 0)),
    )(*([out_lm] * (L * F)))
    return out


@jax.jit
def _prep(x, table):
    return (x[:, 0], x[:, 1], x[:, 2],
            table[:, :, 0].reshape(L * T), table[:, :, 1].reshape(L * T))


def kernel(x, table):
    xa, xb, xc, taba, tabb = _prep(x, table)
    return _encode(xa, xb, xc, taba, tabb)
